# Initial kernel scaffold; baseline (speedup 1.0000x reference)
import jax
import jax.numpy as jnp
from jax.experimental import pallas as pl


def kernel(lt_tokens, mt_tokens, st_tokens, emb_table, W_event, b_event, W_pred, b_pred):
    raise NotImplementedError("write your pallas kernel here")


# speedup vs baseline: 1.3653x; 1.3653x over previous
import functools

import jax
import jax.numpy as jnp
from jax import lax
from jax.experimental import pallas as pl
from jax.experimental.pallas import tpu as pltpu
from jax.experimental.pallas import tpu_sc as plsc

VOCAB = 1000000
DW = 64
DE = 128
B = 4096
L = 20

NC = 2
NS = 16
NW = NC * NS

SENTS = 3 * B * 3
SENT_PER_W = SENTS // NW
CHUNK_S = 24
NCHUNK = SENT_PER_W // CHUNK_S
TOK_PER_CHUNK = CHUNK_S * L
GATHER_ROWS = 120
NG = TOK_PER_CHUNK // GATHER_ROWS
IDX_ROWS = NCHUNK * NG


def _pool_body(table, idx, out, idx_v, rows_v, acc_v,
               sem_idx, sem_g0, sem_g1, sem_o0, sem_o1):
  c = lax.axis_index("c")
  s = lax.axis_index("s")
  wid = s * NC + c
  base0 = wid * SENT_PER_W
  sem_g = (sem_g0, sem_g1)
  sem_o = (sem_o0, sem_o1)

  pltpu.async_copy(idx.at[wid], idx_v, sem_idx).wait()

  def start_gathers(chunk, b):
    for j in range(NG):
      pltpu.async_copy(
          table.at[idx_v.at[chunk * NG + j]],
          rows_v.at[b, pl.ds(j * GATHER_ROWS, GATHER_ROWS)],
          sem_g[b])

  start_gathers(0, 0)
  start_gathers(1, 1)

  def accumulate(b):
    def seg(si, carry):
      tok0 = si * L
      for d in range(DW // 16):
        a = rows_v[b, tok0, pl.ds(d * 16, 16)]
        for t in range(1, L):
          a = a + rows_v[b, tok0 + t, pl.ds(d * 16, 16)]
        acc_v[b, si, pl.ds(d * 16, 16)] = a
      return carry
    lax.fori_loop(0, CHUNK_S, seg, 0)

  def process(chunk, b):
    pltpu.make_async_copy(
        table.at[pl.ds(0, TOK_PER_CHUNK)], rows_v.at[b], sem_g[b]).wait()

    @pl.when(chunk >= 2)
    def _():
      pltpu.make_async_copy(
          acc_v.at[b],
          out.at[pl.ds(base0 + (chunk - 2) * CHUNK_S, CHUNK_S)],
          sem_o[b]).wait()

    accumulate(b)

    pltpu.async_copy(
        acc_v.at[b],
        out.at[pl.ds(base0 + chunk * CHUNK_S, CHUNK_S)],
        sem_o[b])

    @pl.when(chunk + 2 < NCHUNK)
    def _():
      start_gathers(chunk + 2, b)

  def outer(g, carry):
    process(2 * g, 0)
    process(2 * g + 1, 1)
    return carry
  lax.fori_loop(0, NCHUNK // 2, outer, 0)

  for b in range(2):
    chunk = NCHUNK - 2 + b
    pltpu.make_async_copy(
        acc_v.at[b],
        out.at[pl.ds(base0 + chunk * CHUNK_S, CHUNK_S)],
        sem_o[b]).wait()


def _make_pool():
  mesh = plsc.VectorSubcoreMesh(
      core_axis_name="c", subcore_axis_name="s",
      num_cores=NC, num_subcores=NS)
  return pl.kernel(
      _pool_body,
      out_type=jax.ShapeDtypeStruct((SENTS, DW), jnp.float32),
      mesh=mesh,
      compiler_params=pltpu.CompilerParams(use_tc_tiling_on_sc=False),
      scratch_types=[
          pltpu.VMEM((IDX_ROWS, GATHER_ROWS), jnp.int32),
          pltpu.VMEM((2, TOK_PER_CHUNK, DW), jnp.float32),
          pltpu.VMEM((2, CHUNK_S, DW), jnp.float32),
          pltpu.SemaphoreType.DMA,
          pltpu.SemaphoreType.DMA,
          pltpu.SemaphoreType.DMA,
          pltpu.SemaphoreType.DMA,
          pltpu.SemaphoreType.DMA,
      ])


def _mlp_body(lt_ref, mt_ref, st_ref, we_ref, be_ref, wp_ref, bp_ref, out_ref):
  scale = jnp.float32(1.0 / L)

  def ev(x_ref, k):
    x = x_ref[...] * scale
    h = jnp.tanh(
        jnp.dot(x, we_ref[...], preferred_element_type=jnp.float32)
        + be_ref[...])
    wp = wp_ref[k * DE:(k + 1) * DE, :]
    return jnp.dot(h, wp, preferred_element_type=jnp.float32)

  out_ref[...] = ev(lt_ref, 0) + ev(mt_ref, 1) + ev(st_ref, 2) + bp_ref[...]


def _mlp(lt_s, mt_s, st_s, W_event, b_event, W_pred, b_pred):
  bb = 512
  grid = (B // bb,)
  x_spec = pl.BlockSpec((bb, 3 * DW), lambda i: (i, 0))
  return pl.pallas_call(
      _mlp_body,
      grid=grid,
      in_specs=[
          x_spec, x_spec, x_spec,
          pl.BlockSpec((3 * DW, DE), lambda i: (0, 0)),
          pl.BlockSpec((1, DE), lambda i: (0, 0)),
          pl.BlockSpec((3 * DE, 1), lambda i: (0, 0)),
          pl.BlockSpec((1, 1), lambda i: (0, 0)),
      ],
      out_specs=pl.BlockSpec((bb, 1), lambda i: (i, 0)),
      out_shape=jax.ShapeDtypeStruct((B, 1), jnp.float32),
  )(lt_s, mt_s, st_s, W_event, b_event.reshape(1, DE), W_pred,
    b_pred.reshape(1, 1))


def kernel(lt_tokens, mt_tokens, st_tokens, emb_table, W_event, b_event,
           W_pred, b_pred):
  toks = jnp.stack([lt_tokens, mt_tokens, st_tokens])
  idx = toks.astype(jnp.int32).reshape(NW, IDX_ROWS, GATHER_ROWS)
  sums = _make_pool()(emb_table, idx)
  s3 = sums.reshape(3, B, 3 * DW)
  return _mlp(s3[0], s3[1], s3[2], W_event, b_event, W_pred, b_pred)
